# baseline (speedup 1.0000x reference)
import functools

import jax
import jax.numpy as jnp
from jax import lax
from jax.experimental import pallas as pl
from jax.experimental.pallas import tpu as pltpu
from jax.experimental.pallas import tpu_sc as plsc

_N_TOK = 8192
_N_CODES = 8192
_DIM = 32
_TM = 512
_TK = 2048

_SC_CORES = 2
_SC_SUBCORES = 16
_SC_WORKERS = _SC_CORES * _SC_SUBCORES
_GATHER_W = 128


def _nearest_code_kernel(flat_ref, cbt_ref, a2_ref, b2_ref, idx_ref):
    flat = flat_ref[...]
    a2 = a2_ref[...]
    best_val = None
    best_idx = None
    for ci in range(_N_CODES // _TK):
        off = ci * _TK
        cbt_c = cbt_ref[:, off:off + _TK]
        dot = lax.dot_general(flat, cbt_c, (((1,), (0,)), ((), ())),
                              preferred_element_type=jnp.float32)
        dist = jnp.sqrt(jnp.clip(a2 + b2_ref[:, off:off + _TK] - 2.0 * dot,
                                 0.0, None))
        m = jnp.min(dist, axis=1, keepdims=True)
        iota = lax.broadcasted_iota(jnp.int32, (_TM, _TK), 1) + off
        cand = jnp.where(dist == m, iota, jnp.int32(_N_CODES))
        arg = jnp.min(cand, axis=1, keepdims=True)
        m_bf = m.astype(jnp.bfloat16).astype(jnp.float32)
        if best_val is None:
            best_val, best_idx = m_bf, arg
        else:
            take = m < best_val
            best_val = jnp.where(take, m_bf, best_val)
            best_idx = jnp.where(take, arg, best_idx)
    idx_ref[...] = best_idx


def _nearest_codes(flat_bf, cbt_bf, a2, b2):
    return pl.pallas_call(
        _nearest_code_kernel,
        grid=(_N_TOK // _TM,),
        in_specs=[
            pl.BlockSpec((_TM, _DIM), lambda i: (i, 0)),
            pl.BlockSpec((_DIM, _N_CODES), lambda i: (0, 0)),
            pl.BlockSpec((_TM, 1), lambda i: (i, 0)),
            pl.BlockSpec((1, _N_CODES), lambda i: (0, 0)),
        ],
        out_specs=pl.BlockSpec((_TM, 1), lambda i: (i, 0)),
        out_shape=jax.ShapeDtypeStruct((_N_TOK, 1), jnp.int32),
    )(flat_bf, cbt_bf, a2, b2)


def _sc_gather(codebook_padded, idx):
    mesh = plsc.VectorSubcoreMesh(core_axis_name="c", subcore_axis_name="s")
    b_per_w = _N_TOK // _SC_WORKERS

    @functools.partial(
        pl.kernel, mesh=mesh,
        out_type=jax.ShapeDtypeStruct((_N_TOK, _GATHER_W), jnp.float32),
        scratch_types=[
            pltpu.VMEM((b_per_w,), jnp.int32),
            pltpu.VMEM((b_per_w, _GATHER_W), jnp.float32),
            pltpu.SemaphoreType.DMA,
        ],
    )
    def k(table_hbm, idx_hbm, out_hbm, idx_v, rows_v, sem):
        wid = lax.axis_index("s") * _SC_CORES + lax.axis_index("c")
        base = wid * b_per_w
        pltpu.sync_copy(idx_hbm.at[pl.ds(base, b_per_w)], idx_v)
        pltpu.async_copy(table_hbm.at[idx_v], rows_v, sem).wait()
        pltpu.sync_copy(rows_v, out_hbm.at[pl.ds(base, b_per_w)])

    return k(codebook_padded, idx)


def kernel(hidden_states, codebook):
    hs = jnp.transpose(hidden_states, (0, 2, 3, 1))
    flat = hs.reshape(-1, codebook.shape[1])
    a2 = jnp.sum(flat * flat, axis=1, keepdims=True)
    b2 = jnp.sum(codebook * codebook, axis=1)[None, :]
    flat_bf = flat.astype(jnp.bfloat16)
    cbt_bf = codebook.astype(jnp.bfloat16).T
    idx = _nearest_codes(flat_bf, cbt_bf, a2, b2).reshape(_N_TOK)
    cb_pad = jnp.pad(codebook, ((0, 0), (0, _GATHER_W - _DIM)))
    z_q = _sc_gather(cb_pad, idx)[:, :_DIM]
    z_q = jnp.transpose(z_q.reshape(hs.shape), (0, 3, 1, 2))
    return (z_q, idx.reshape(hidden_states.shape[0], -1))


# speedup vs baseline: 1.0685x; 1.0685x over previous
import functools

import jax
import jax.numpy as jnp
from jax import lax
from jax.experimental import pallas as pl
from jax.experimental.pallas import tpu as pltpu
from jax.experimental.pallas import tpu_sc as plsc

_N_TOK = 8192
_N_CODES = 8192
_DIM = 32
_TM = 512
_TK = 2048

_SC_CORES = 2
_SC_SUBCORES = 16
_SC_WORKERS = _SC_CORES * _SC_SUBCORES
_GATHER_W = 128


def _nearest_code_kernel(flat_ref, cbt_ref, a2_ref, b2_ref, idx_ref):
    flat = flat_ref[...]
    a2 = a2_ref[...]
    best_val = None
    best_idx = None
    for ci in range(_N_CODES // _TK):
        off = ci * _TK
        cbt_c = cbt_ref[:, off:off + _TK]
        dot = lax.dot_general(flat, cbt_c, (((1,), (0,)), ((), ())),
                              preferred_element_type=jnp.float32)
        dist = jnp.sqrt(jnp.clip(a2 + b2_ref[:, off:off + _TK] - 2.0 * dot,
                                 0.0, None))
        best_v = dist[:, 0:128]
        best_c = jnp.zeros((_TM, 128), jnp.int32)
        for c in range(1, _TK // 128):
            v = dist[:, c * 128:(c + 1) * 128]
            lt = v < best_v
            best_v = jnp.where(lt, v, best_v)
            best_c = jnp.where(lt, jnp.int32(c), best_c)
        m = jnp.min(best_v, axis=1, keepdims=True)
        lane = lax.broadcasted_iota(jnp.int32, (_TM, 128), 1)
        idxs = best_c * 128 + lane + off
        cand = jnp.where(best_v == m, idxs, jnp.int32(_N_CODES))
        arg = jnp.min(cand, axis=1, keepdims=True)
        m_bf = m.astype(jnp.bfloat16).astype(jnp.float32)
        if best_val is None:
            best_val, best_idx = m_bf, arg
        else:
            take = m < best_val
            best_val = jnp.where(take, m_bf, best_val)
            best_idx = jnp.where(take, arg, best_idx)
    idx_ref[...] = best_idx


def _nearest_codes(flat_bf, cbt_bf, a2, b2):
    return pl.pallas_call(
        _nearest_code_kernel,
        grid=(_N_TOK // _TM,),
        in_specs=[
            pl.BlockSpec((_TM, _DIM), lambda i: (i, 0)),
            pl.BlockSpec((_DIM, _N_CODES), lambda i: (0, 0)),
            pl.BlockSpec((_TM, 1), lambda i: (i, 0)),
            pl.BlockSpec((1, _N_CODES), lambda i: (0, 0)),
        ],
        out_specs=pl.BlockSpec((_TM, 1), lambda i: (i, 0)),
        out_shape=jax.ShapeDtypeStruct((_N_TOK, 1), jnp.int32),
    )(flat_bf, cbt_bf, a2, b2)


def _sc_gather(codebook_padded, idx):
    mesh = plsc.VectorSubcoreMesh(core_axis_name="c", subcore_axis_name="s")
    b_per_w = _N_TOK // _SC_WORKERS

    @functools.partial(
        pl.kernel, mesh=mesh,
        out_type=jax.ShapeDtypeStruct((_N_TOK, _GATHER_W), jnp.float32),
        scratch_types=[
            pltpu.VMEM((b_per_w,), jnp.int32),
            pltpu.VMEM((b_per_w, _GATHER_W), jnp.float32),
            pltpu.SemaphoreType.DMA,
        ],
    )
    def k(table_hbm, idx_hbm, out_hbm, idx_v, rows_v, sem):
        wid = lax.axis_index("s") * _SC_CORES + lax.axis_index("c")
        base = wid * b_per_w
        pltpu.sync_copy(idx_hbm.at[pl.ds(base, b_per_w)], idx_v)
        pltpu.async_copy(table_hbm.at[idx_v], rows_v, sem).wait()
        pltpu.sync_copy(rows_v, out_hbm.at[pl.ds(base, b_per_w)])

    return k(codebook_padded, idx)


def kernel(hidden_states, codebook):
    hs = jnp.transpose(hidden_states, (0, 2, 3, 1))
    flat = hs.reshape(-1, codebook.shape[1])
    a2 = jnp.sum(flat * flat, axis=1, keepdims=True)
    b2 = jnp.sum(codebook * codebook, axis=1)[None, :]
    flat_bf = flat.astype(jnp.bfloat16)
    cbt_bf = codebook.astype(jnp.bfloat16).T
    idx = _nearest_codes(flat_bf, cbt_bf, a2, b2).reshape(_N_TOK)
    cb_pad = jnp.pad(codebook, ((0, 0), (0, _GATHER_W - _DIM)))
    z_q = _sc_gather(cb_pad, idx)[:, :_DIM]
    z_q = jnp.transpose(z_q.reshape(hs.shape), (0, 3, 1, 2))
    return (z_q, idx.reshape(hidden_states.shape[0], -1))


# speedup vs baseline: 1.0944x; 1.0243x over previous
import functools

import jax
import jax.numpy as jnp
from jax import lax
from jax.experimental import pallas as pl
from jax.experimental.pallas import tpu as pltpu
from jax.experimental.pallas import tpu_sc as plsc

_N_TOK = 8192
_N_CODES = 8192
_DIM = 32
_TM = 1024
_TK = 2048

_SC_CORES = 2
_SC_SUBCORES = 16
_SC_WORKERS = _SC_CORES * _SC_SUBCORES
_GATHER_W = 128


def _nearest_code_kernel(flat_ref, cbt_ref, a2_ref, b2_ref, idx_ref):
    flat = flat_ref[...]
    a2 = a2_ref[...]
    best_val = None
    best_idx = None
    for ci in range(_N_CODES // _TK):
        off = ci * _TK
        cbt_c = cbt_ref[:, off:off + _TK]
        dot = lax.dot_general(flat, cbt_c, (((1,), (0,)), ((), ())),
                              preferred_element_type=jnp.float32)
        dist = jnp.sqrt(jnp.clip(a2 + b2_ref[:, off:off + _TK] - 2.0 * dot,
                                 0.0, None))
        best_v = dist[:, 0:128]
        best_c = jnp.zeros((_TM, 128), jnp.int32)
        for c in range(1, _TK // 128):
            v = dist[:, c * 128:(c + 1) * 128]
            lt = v < best_v
            best_v = jnp.where(lt, v, best_v)
            best_c = jnp.where(lt, jnp.int32(c), best_c)
        m = jnp.min(best_v, axis=1, keepdims=True)
        lane = lax.broadcasted_iota(jnp.int32, (_TM, 128), 1)
        idxs = best_c * 128 + lane + off
        cand = jnp.where(best_v == m, idxs, jnp.int32(_N_CODES))
        arg = jnp.min(cand, axis=1, keepdims=True)
        m_bf = m.astype(jnp.bfloat16).astype(jnp.float32)
        if best_val is None:
            best_val, best_idx = m_bf, arg
        else:
            take = m < best_val
            best_val = jnp.where(take, m_bf, best_val)
            best_idx = jnp.where(take, arg, best_idx)
    idx_ref[...] = best_idx


def _nearest_codes(flat_bf, cbt_bf, a2, b2):
    return pl.pallas_call(
        _nearest_code_kernel,
        grid=(_N_TOK // _TM,),
        in_specs=[
            pl.BlockSpec((_TM, _DIM), lambda i: (i, 0)),
            pl.BlockSpec((_DIM, _N_CODES), lambda i: (0, 0)),
            pl.BlockSpec((_TM, 1), lambda i: (i, 0)),
            pl.BlockSpec((1, _N_CODES), lambda i: (0, 0)),
        ],
        out_specs=pl.BlockSpec((_TM, 1), lambda i: (i, 0)),
        out_shape=jax.ShapeDtypeStruct((_N_TOK, 1), jnp.int32),
    )(flat_bf, cbt_bf, a2, b2)


def _sc_gather(codebook_padded, idx):
    mesh = plsc.VectorSubcoreMesh(core_axis_name="c", subcore_axis_name="s")
    b_per_w = _N_TOK // _SC_WORKERS

    @functools.partial(
        pl.kernel, mesh=mesh,
        out_type=jax.ShapeDtypeStruct((_N_TOK, _GATHER_W), jnp.float32),
        scratch_types=[
            pltpu.VMEM((b_per_w,), jnp.int32),
            pltpu.VMEM((b_per_w, _GATHER_W), jnp.float32),
            pltpu.SemaphoreType.DMA,
        ],
    )
    def k(table_hbm, idx_hbm, out_hbm, idx_v, rows_v, sem):
        wid = lax.axis_index("s") * _SC_CORES + lax.axis_index("c")
        base = wid * b_per_w
        pltpu.sync_copy(idx_hbm.at[pl.ds(base, b_per_w)], idx_v)
        pltpu.async_copy(table_hbm.at[idx_v], rows_v, sem).wait()
        pltpu.sync_copy(rows_v, out_hbm.at[pl.ds(base, b_per_w)])

    return k(codebook_padded, idx)


def kernel(hidden_states, codebook):
    hs = jnp.transpose(hidden_states, (0, 2, 3, 1))
    flat = hs.reshape(-1, codebook.shape[1])
    a2 = jnp.sum(flat * flat, axis=1, keepdims=True)
    b2 = jnp.sum(codebook * codebook, axis=1)[None, :]
    flat_bf = flat.astype(jnp.bfloat16)
    cbt_bf = codebook.astype(jnp.bfloat16).T
    idx = _nearest_codes(flat_bf, cbt_bf, a2, b2).reshape(_N_TOK)
    cb_pad = jnp.pad(codebook, ((0, 0), (0, _GATHER_W - _DIM)))
    z_q = _sc_gather(cb_pad, idx)[:, :_DIM]
    z_q = jnp.transpose(z_q.reshape(hs.shape), (0, 3, 1, 2))
    return (z_q, idx.reshape(hidden_states.shape[0], -1))
